# baseline (speedup 1.0000x reference)
import functools

import jax
import jax.numpy as jnp
from jax import lax
from jax.experimental import pallas as pl
from jax.experimental.pallas import tpu as pltpu
from jax.experimental.pallas import tpu_sc as plsc

B, S, D = 2, 2048, 2048
E, TK, H = 8, 2, 1024
T = B * S
NPAIR = T * TK
BLK = 256
P = NPAIR + E * BLK
NB = P // BLK
NBPAD = 48
HB = 256
NH = H // HB

NC, NS = 2, 16
NW = NC * NS
PW = NPAIR // NW
TW = T // NW

_MESH = plsc.VectorSubcoreMesh(
    core_axis_name="c", subcore_axis_name="s", num_cores=NC, num_subcores=NS)


def _wid():
    return lax.axis_index("s") * NC + lax.axis_index("c")


def _router_body(x_ref, g_ref, topi_ref, topv_ref):
    xb = x_ref[...]
    gw = g_ref[...]
    logits = lax.dot_general(xb, gw, (((1,), (1,)), ((), ())),
                             preferred_element_type=jnp.float32)
    m = jnp.max(logits, axis=-1, keepdims=True)
    p = jnp.exp(logits - m)
    p = p / jnp.sum(p, axis=-1, keepdims=True)
    i8 = lax.broadcasted_iota(jnp.int32, p.shape, 1)
    m1 = jnp.max(p, axis=-1, keepdims=True)
    i1 = jnp.min(jnp.where(p == m1, i8, E + 1), axis=-1, keepdims=True)
    p2 = jnp.where(i8 == i1, -1.0, p)
    m2 = jnp.max(p2, axis=-1, keepdims=True)
    i2 = jnp.min(jnp.where(p2 == m2, i8, E + 1), axis=-1, keepdims=True)
    s = m1 + m2
    topi_ref[...] = jnp.concatenate([i1, i2], axis=1)
    topv_ref[...] = jnp.concatenate([m1 / s, m2 / s], axis=1)


def _router(x2, gate_w):
    rb = 256
    return pl.pallas_call(
        _router_body,
        grid=(T // rb,),
        in_specs=[
            pl.BlockSpec((rb, D), lambda i: (i, 0)),
            pl.BlockSpec((E, D), lambda i: (0, 0)),
        ],
        out_specs=[
            pl.BlockSpec((rb, TK), lambda i: (i, 0)),
            pl.BlockSpec((rb, TK), lambda i: (i, 0)),
        ],
        out_shape=[
            jax.ShapeDtypeStruct((T, TK), jnp.int32),
            jax.ShapeDtypeStruct((T, TK), jnp.float32),
        ],
    )(x2, gate_w)


_XCH = 16

def _k2_body(topi_hbm, x_hbm, pos_hbm, blk_hbm, xs_hbm,
             buf, base_v, cur_v, pos_v, offblk_v, blk_v,
             xrows0, xrows1, idx_e0, idx_o0, idx_e1, idx_o1, sem):
    w = _wid()
    pltpu.sync_copy(topi_hbm, buf)
    zeros = jnp.zeros((16,), jnp.int32)
    ones = jnp.ones((16,), jnp.int32)
    base_v[...] = zeros
    cur_v[...] = zeros
    wstart = w * PW
    ii = lax.iota(jnp.int32, 16)
    for i in range(NPAIR // 16):
        v = buf[pl.ds(i * 16, 16)]
        plsc.addupdate_scatter(cur_v, [v], ones)
        inc = jnp.where(ii + i * 16 < wstart, 1, 0)
        plsc.addupdate_scatter(base_v, [v], inc)
    tot = cur_v[...]
    prior = base_v[...]
    pad = (tot + (BLK - 1)) & ~(BLK - 1)
    offpad = plsc.cumsum(pad) - pad
    base_v[...] = offpad + prior
    cur_v[...] = zeros
    for i in range(PW // 16):
        v = buf[pl.ds(wstart + i * 16, 16)]
        gb = plsc.load_gather(base_v, [v])
        gc = plsc.load_gather(cur_v, [v])
        plsc.addupdate_scatter(cur_v, [v], ones)
        rank = zeros
        for e in range(E):
            m = v == e
            r = plsc.cumsum(jnp.where(m, 1, 0))
            rank = jnp.where(m, r - 1, rank)
        pos_v[pl.ds(i * 16, 16)] = gb + gc + rank
    pltpu.sync_copy(pos_v, pos_hbm.at[pl.ds(wstart, PW)])

    @pl.when(w == 0)
    def _():
        offblk_v[...] = offpad // BLK
        for c in range(NBPAD // 16):
            bvec = lax.iota(jnp.int32, 16) + c * 16
            acc = jnp.zeros((16,), jnp.int32)
            for e in range(1, E):
                s_e = plsc.load_gather(offblk_v, [jnp.full((16,), e, jnp.int32)])
                acc = acc + jnp.where(bvec >= s_e, 1, 0)
            blk_v[pl.ds(c * 16, 16)] = acc
        pltpu.sync_copy(blk_v, blk_hbm)

    xrows = (xrows0, xrows1)
    idx_e = (idx_e0, idx_e1)
    idx_o = (idx_o0, idx_o1)
    nch = TW // _XCH
    cps = [None, None]

    def issue(c, s):
        tok0 = w * TW + c * _XCH
        pltpu.sync_copy(x_hbm.at[pl.ds(tok0, _XCH)], xrows[s])
        idx_e[s][...] = plsc.load_gather(pos_v, [c * 2 * _XCH + 2 * ii])
        idx_o[s][...] = plsc.load_gather(pos_v, [c * 2 * _XCH + 2 * ii + 1])
        return (pltpu.async_copy(xrows[s], xs_hbm.at[idx_e[s]], sem),
                pltpu.async_copy(xrows[s], xs_hbm.at[idx_o[s]], sem))

    cps[0] = issue(0, 0)
    for c in range(1, nch):
        s = c % 2
        cps[s] = issue(c, s)
        cps[1 - s][0].wait()
        cps[1 - s][1].wait()
    cps[(nch - 1) % 2][0].wait()
    cps[(nch - 1) % 2][1].wait()


def _dispatch_scatter(topi_flat, x2):
    return pl.kernel(
        _k2_body,
        out_type=[
            jax.ShapeDtypeStruct((NPAIR,), jnp.int32),
            jax.ShapeDtypeStruct((NBPAD,), jnp.int32),
            jax.ShapeDtypeStruct((P, D), jnp.float32),
        ],
        mesh=_MESH,
        compiler_params=pltpu.CompilerParams(needs_layout_passes=False),
        scratch_types=[
            pltpu.VMEM((NPAIR,), jnp.int32),
            pltpu.VMEM((16,), jnp.int32),
            pltpu.VMEM((16,), jnp.int32),
            pltpu.VMEM((PW,), jnp.int32),
            pltpu.VMEM((16,), jnp.int32),
            pltpu.VMEM((NBPAD,), jnp.int32),
            pltpu.VMEM((_XCH, D), jnp.float32),
            pltpu.VMEM((_XCH, D), jnp.float32),
            pltpu.VMEM((16,), jnp.int32),
            pltpu.VMEM((16,), jnp.int32),
            pltpu.VMEM((16,), jnp.int32),
            pltpu.VMEM((16,), jnp.int32),
            pltpu.SemaphoreType.DMA,
        ],
    )(topi_flat, x2)


def _ffn_body(be_ref, xs_ref, wg_ref, w1_ref, w2_ref, eo_ref):
    xb = xs_ref[...]
    g = lax.dot_general(xb, wg_ref[0], (((1,), (1,)), ((), ())),
                        preferred_element_type=jnp.float32)
    a = lax.dot_general(xb, w1_ref[0], (((1,), (1,)), ((), ())),
                        preferred_element_type=jnp.float32)
    h = a * (g * jax.nn.sigmoid(g))
    eo_ref[...] = lax.dot_general(h, w2_ref[0], (((1,), (1,)), ((), ())),
                                  preferred_element_type=jnp.float32)


def _ffn(blkexp, xs, wg, w1, w2):
    grid_spec = pltpu.PrefetchScalarGridSpec(
        num_scalar_prefetch=1,
        grid=(NB,),
        in_specs=[
            pl.BlockSpec((BLK, D), lambda b, be: (b, 0)),
            pl.BlockSpec((1, H, D), lambda b, be: (be[b], 0, 0)),
            pl.BlockSpec((1, H, D), lambda b, be: (be[b], 0, 0)),
            pl.BlockSpec((1, D, H), lambda b, be: (be[b], 0, 0)),
        ],
        out_specs=pl.BlockSpec((BLK, D), lambda b, be: (b, 0)),
    )
    return pl.pallas_call(
        _ffn_body,
        grid_spec=grid_spec,
        out_shape=jax.ShapeDtypeStruct((P, D), jnp.float32),
        compiler_params=pltpu.CompilerParams(
            dimension_semantics=("arbitrary",),
            vmem_limit_bytes=100 * 1024 * 1024),
    )(blkexp, xs, wg, w1, w2)


_CCH = 8

def _comb_body(eo_hbm, pos_hbm, tv_hbm, out_hbm, pos_v, tv_v,
               idx0, idx1, rows0, rows1, orow, sem):
    w = _wid()
    pltpu.sync_copy(pos_hbm.at[pl.ds(w * PW, PW)], pos_v)
    pltpu.sync_copy(tv_hbm.at[pl.ds(w * PW, PW)], tv_v)
    idx = (idx0, idx1)
    rows = (rows0, rows1)
    nch = TW // _CCH
    cps = [None, None]

    def issue(c, s):
        idx[s][...] = pos_v[pl.ds(c * 2 * _CCH, 2 * _CCH)]
        return pltpu.async_copy(eo_hbm.at[idx[s]], rows[s], sem)

    def compute(c, s):
        rw = rows[s]
        wv = tv_v[pl.ds(c * 2 * _CCH, 2 * _CCH)]
        ws = [wv[j] for j in range(2 * _CCH)]

        def body_r(r, _):
            off = r * 16
            for i in range(_CCH):
                a = rw[2 * i, pl.ds(off, 16)]
                b = rw[2 * i + 1, pl.ds(off, 16)]
                orow[i, pl.ds(off, 16)] = a * ws[2 * i] + b * ws[2 * i + 1]
            return 0

        lax.fori_loop(0, D // 16, body_r, 0, unroll=2)
        pltpu.sync_copy(orow, out_hbm.at[pl.ds(w * TW + c * _CCH, _CCH)])

    cps[0] = issue(0, 0)
    for c in range(1, nch):
        s = c % 2
        cps[s] = issue(c, s)
        cps[1 - s].wait()
        compute(c - 1, 1 - s)
    cps[(nch - 1) % 2].wait()
    compute(nch - 1, (nch - 1) % 2)


def _combine(eo, pos, tv_flat):
    return pl.kernel(
        _comb_body,
        out_type=jax.ShapeDtypeStruct((T, D), jnp.float32),
        mesh=_MESH,
        compiler_params=pltpu.CompilerParams(needs_layout_passes=False),
        scratch_types=[
            pltpu.VMEM((PW,), jnp.int32),
            pltpu.VMEM((PW,), jnp.float32),
            pltpu.VMEM((2 * _CCH,), jnp.int32),
            pltpu.VMEM((2 * _CCH,), jnp.int32),
            pltpu.VMEM((2 * _CCH, D), jnp.float32),
            pltpu.VMEM((2 * _CCH, D), jnp.float32),
            pltpu.VMEM((_CCH, D), jnp.float32),
            pltpu.SemaphoreType.DMA,
        ],
    )(eo, pos, tv_flat)


def kernel(x, gate_w, w1, wg, w2):
    x2 = x.reshape(T, D)
    topi, topv = _router(x2, gate_w)
    topi_flat = topi.reshape(NPAIR)
    tv_flat = topv.reshape(NPAIR)
    pos, blkexp, xs = _dispatch_scatter(topi_flat, x2)
    eo = _ffn(blkexp, xs, wg, w1, w2)
    out2 = _combine(eo, pos, tv_flat)
    return out2.reshape(B, S, D)


# speedup vs baseline: 1.0017x; 1.0017x over previous
import functools

import jax
import jax.numpy as jnp
from jax import lax
from jax.experimental import pallas as pl
from jax.experimental.pallas import tpu as pltpu
from jax.experimental.pallas import tpu_sc as plsc

B, S, D = 2, 2048, 2048
E, TK, H = 8, 2, 1024
T = B * S
NPAIR = T * TK
BLK = 256
P = NPAIR + E * BLK
NB = P // BLK
NBPAD = 48
HB = 256
NH = H // HB

NC, NS = 2, 16
NW = NC * NS
PW = NPAIR // NW
TW = T // NW

_MESH = plsc.VectorSubcoreMesh(
    core_axis_name="c", subcore_axis_name="s", num_cores=NC, num_subcores=NS)


def _wid():
    return lax.axis_index("s") * NC + lax.axis_index("c")


def _router_body(x_ref, g_ref, topi_ref, topv_ref):
    xb = x_ref[...]
    gw = g_ref[...]
    logits = lax.dot_general(xb, gw, (((1,), (1,)), ((), ())),
                             preferred_element_type=jnp.float32)
    m = jnp.max(logits, axis=-1, keepdims=True)
    p = jnp.exp(logits - m)
    p = p / jnp.sum(p, axis=-1, keepdims=True)
    i8 = lax.broadcasted_iota(jnp.int32, p.shape, 1)
    m1 = jnp.max(p, axis=-1, keepdims=True)
    i1 = jnp.min(jnp.where(p == m1, i8, E + 1), axis=-1, keepdims=True)
    p2 = jnp.where(i8 == i1, -1.0, p)
    m2 = jnp.max(p2, axis=-1, keepdims=True)
    i2 = jnp.min(jnp.where(p2 == m2, i8, E + 1), axis=-1, keepdims=True)
    s = m1 + m2
    topi_ref[...] = jnp.concatenate([i1, i2], axis=1)
    topv_ref[...] = jnp.concatenate([m1 / s, m2 / s], axis=1)


def _router(x2, gate_w):
    rb = 256
    return pl.pallas_call(
        _router_body,
        grid=(T // rb,),
        in_specs=[
            pl.BlockSpec((rb, D), lambda i: (i, 0)),
            pl.BlockSpec((E, D), lambda i: (0, 0)),
        ],
        out_specs=[
            pl.BlockSpec((rb, TK), lambda i: (i, 0)),
            pl.BlockSpec((rb, TK), lambda i: (i, 0)),
        ],
        out_shape=[
            jax.ShapeDtypeStruct((T, TK), jnp.int32),
            jax.ShapeDtypeStruct((T, TK), jnp.float32),
        ],
    )(x2, gate_w)


_XCH = 16

def _k2_body(topi_hbm, x_hbm, pos_hbm, blk_hbm, xs_hbm,
             buf, base_v, cur_v, pos_v, offblk_v, blk_v,
             xrows0, xrows1, idx_e0, idx_o0, idx_e1, idx_o1, sem):
    w = _wid()
    pltpu.sync_copy(topi_hbm, buf)
    zeros = jnp.zeros((16,), jnp.int32)
    ones = jnp.ones((16,), jnp.int32)
    base_v[...] = zeros
    cur_v[...] = zeros
    wstart = w * PW
    ii = lax.iota(jnp.int32, 16)
    for i in range(NPAIR // 16):
        v = buf[pl.ds(i * 16, 16)]
        plsc.addupdate_scatter(cur_v, [v], ones)
        inc = jnp.where(ii + i * 16 < wstart, 1, 0)
        plsc.addupdate_scatter(base_v, [v], inc)
    tot = cur_v[...]
    prior = base_v[...]
    pad = (tot + (BLK - 1)) & ~(BLK - 1)
    offpad = plsc.cumsum(pad) - pad
    base_v[...] = offpad + prior
    cur_v[...] = zeros
    for i in range(PW // 16):
        v = buf[pl.ds(wstart + i * 16, 16)]
        gb = plsc.load_gather(base_v, [v])
        gc = plsc.load_gather(cur_v, [v])
        plsc.addupdate_scatter(cur_v, [v], ones)
        rank = zeros
        for e in range(E):
            m = v == e
            r = plsc.cumsum(jnp.where(m, 1, 0))
            rank = jnp.where(m, r - 1, rank)
        pos_v[pl.ds(i * 16, 16)] = gb + gc + rank
    pltpu.sync_copy(pos_v, pos_hbm.at[pl.ds(wstart, PW)])

    @pl.when(w == 0)
    def _():
        offblk_v[...] = offpad // BLK
        for c in range(NBPAD // 16):
            bvec = lax.iota(jnp.int32, 16) + c * 16
            acc = jnp.zeros((16,), jnp.int32)
            for e in range(1, E):
                s_e = plsc.load_gather(offblk_v, [jnp.full((16,), e, jnp.int32)])
                acc = acc + jnp.where(bvec >= s_e, 1, 0)
            blk_v[pl.ds(c * 16, 16)] = acc
        pltpu.sync_copy(blk_v, blk_hbm)

    xrows = (xrows0, xrows1)
    idx_e = (idx_e0, idx_e1)
    idx_o = (idx_o0, idx_o1)
    nch = TW // _XCH
    cps = [None, None]

    def issue(c, s):
        tok0 = w * TW + c * _XCH
        pltpu.sync_copy(x_hbm.at[pl.ds(tok0, _XCH)], xrows[s])
        idx_e[s][...] = plsc.load_gather(pos_v, [c * 2 * _XCH + 2 * ii])
        idx_o[s][...] = plsc.load_gather(pos_v, [c * 2 * _XCH + 2 * ii + 1])
        return (pltpu.async_copy(xrows[s], xs_hbm.at[idx_e[s]], sem),
                pltpu.async_copy(xrows[s], xs_hbm.at[idx_o[s]], sem))

    cps[0] = issue(0, 0)
    for c in range(1, nch):
        s = c % 2
        cps[s] = issue(c, s)
        cps[1 - s][0].wait()
        cps[1 - s][1].wait()
    cps[(nch - 1) % 2][0].wait()
    cps[(nch - 1) % 2][1].wait()


def _dispatch_scatter(topi_flat, x2):
    return pl.kernel(
        _k2_body,
        out_type=[
            jax.ShapeDtypeStruct((NPAIR,), jnp.int32),
            jax.ShapeDtypeStruct((NBPAD,), jnp.int32),
            jax.ShapeDtypeStruct((P, D), jnp.float32),
        ],
        mesh=_MESH,
        compiler_params=pltpu.CompilerParams(needs_layout_passes=False),
        scratch_types=[
            pltpu.VMEM((NPAIR,), jnp.int32),
            pltpu.VMEM((16,), jnp.int32),
            pltpu.VMEM((16,), jnp.int32),
            pltpu.VMEM((PW,), jnp.int32),
            pltpu.VMEM((16,), jnp.int32),
            pltpu.VMEM((NBPAD,), jnp.int32),
            pltpu.VMEM((_XCH, D), jnp.float32),
            pltpu.VMEM((_XCH, D), jnp.float32),
            pltpu.VMEM((16,), jnp.int32),
            pltpu.VMEM((16,), jnp.int32),
            pltpu.VMEM((16,), jnp.int32),
            pltpu.VMEM((16,), jnp.int32),
            pltpu.SemaphoreType.DMA,
        ],
    )(topi_flat, x2)


def _ffn_body(be_ref, xs_ref, wg_ref, w1_ref, w2_ref, eo_ref):
    xb = xs_ref[...]
    g = lax.dot_general(xb, wg_ref[0], (((1,), (1,)), ((), ())),
                        preferred_element_type=jnp.float32)
    a = lax.dot_general(xb, w1_ref[0], (((1,), (1,)), ((), ())),
                        preferred_element_type=jnp.float32)
    h = a * (g * jax.nn.sigmoid(g))
    eo_ref[...] = lax.dot_general(h, w2_ref[0], (((1,), (1,)), ((), ())),
                                  preferred_element_type=jnp.float32)


def _ffn(blkexp, xs, wg, w1, w2):
    grid_spec = pltpu.PrefetchScalarGridSpec(
        num_scalar_prefetch=1,
        grid=(NB,),
        in_specs=[
            pl.BlockSpec((BLK, D), lambda b, be: (b, 0)),
            pl.BlockSpec((1, H, D), lambda b, be: (be[b], 0, 0)),
            pl.BlockSpec((1, H, D), lambda b, be: (be[b], 0, 0)),
            pl.BlockSpec((1, D, H), lambda b, be: (be[b], 0, 0)),
        ],
        out_specs=pl.BlockSpec((BLK, D), lambda b, be: (b, 0)),
    )
    return pl.pallas_call(
        _ffn_body,
        grid_spec=grid_spec,
        out_shape=jax.ShapeDtypeStruct((P, D), jnp.float32),
        compiler_params=pltpu.CompilerParams(
            dimension_semantics=("arbitrary",),
            vmem_limit_bytes=100 * 1024 * 1024),
    )(blkexp, xs, wg, w1, w2)


_CCH = 8

def _comb_body(eo_hbm, pos_hbm, tv_hbm, out_hbm, pos_v, tv_v,
               idx0, idx1, rows0, rows1, orow, sem):
    w = _wid()
    pltpu.sync_copy(pos_hbm.at[pl.ds(w * PW, PW)], pos_v)
    pltpu.sync_copy(tv_hbm.at[pl.ds(w * PW, PW)], tv_v)
    idx = (idx0, idx1)
    rows = (rows0, rows1)
    nch = TW // _CCH
    cps = [None, None]

    def issue(c, s):
        idx[s][...] = pos_v[pl.ds(c * 2 * _CCH, 2 * _CCH)]
        return pltpu.async_copy(eo_hbm.at[idx[s]], rows[s], sem)

    zero16 = jnp.zeros((16,), jnp.float32)

    def compute(c, s):
        rw = rows[s]
        wv = tv_v[pl.ds(c * 2 * _CCH, 2 * _CCH)]
        wb = [wv[j] + zero16 for j in range(2 * _CCH)]

        def body_r(r, _):
            off = r * 16
            for i in range(_CCH):
                a = rw[2 * i, pl.ds(off, 16)]
                b = rw[2 * i + 1, pl.ds(off, 16)]
                orow[i, pl.ds(off, 16)] = a * wb[2 * i] + b * wb[2 * i + 1]
            return 0

        lax.fori_loop(0, D // 16, body_r, 0, unroll=2)
        pltpu.sync_copy(orow, out_hbm.at[pl.ds(w * TW + c * _CCH, _CCH)])

    cps[0] = issue(0, 0)
    for c in range(1, nch):
        s = c % 2
        cps[s] = issue(c, s)
        cps[1 - s].wait()
        compute(c - 1, 1 - s)
    cps[(nch - 1) % 2].wait()
    compute(nch - 1, (nch - 1) % 2)


def _combine(eo, pos, tv_flat):
    return pl.kernel(
        _comb_body,
        out_type=jax.ShapeDtypeStruct((T, D), jnp.float32),
        mesh=_MESH,
        compiler_params=pltpu.CompilerParams(needs_layout_passes=False),
        scratch_types=[
            pltpu.VMEM((PW,), jnp.int32),
            pltpu.VMEM((PW,), jnp.float32),
            pltpu.VMEM((2 * _CCH,), jnp.int32),
            pltpu.VMEM((2 * _CCH,), jnp.int32),
            pltpu.VMEM((2 * _CCH, D), jnp.float32),
            pltpu.VMEM((2 * _CCH, D), jnp.float32),
            pltpu.VMEM((_CCH, D), jnp.float32),
            pltpu.SemaphoreType.DMA,
        ],
    )(eo, pos, tv_flat)


def kernel(x, gate_w, w1, wg, w2):
    x2 = x.reshape(T, D)
    topi, topv = _router(x2, gate_w)
    topi_flat = topi.reshape(NPAIR)
    tv_flat = topv.reshape(NPAIR)
    pos, blkexp, xs = _dispatch_scatter(topi_flat, x2)
    eo = _ffn(blkexp, xs, wg, w1, w2)
    out2 = _combine(eo, pos, tv_flat)
    return out2.reshape(B, S, D)


# speedup vs baseline: 1.0591x; 1.0573x over previous
import functools

import jax
import jax.numpy as jnp
from jax import lax
from jax.experimental import pallas as pl
from jax.experimental.pallas import tpu as pltpu
from jax.experimental.pallas import tpu_sc as plsc

B, S, D = 2, 2048, 2048
E, TK, H = 8, 2, 1024
T = B * S
NPAIR = T * TK
BLK = 256
P = NPAIR + E * BLK
NB = P // BLK
NBPAD = 48
HB = 256
NH = H // HB

NC, NS = 2, 16
NW = NC * NS
PW = NPAIR // NW
TW = T // NW

_MESH = plsc.VectorSubcoreMesh(
    core_axis_name="c", subcore_axis_name="s", num_cores=NC, num_subcores=NS)


def _wid():
    return lax.axis_index("s") * NC + lax.axis_index("c")


def _router_body(x_ref, g_ref, topi_ref, topv_ref):
    xb = x_ref[...]
    gw = g_ref[...]
    logits = lax.dot_general(xb, gw, (((1,), (1,)), ((), ())),
                             preferred_element_type=jnp.float32)
    m = jnp.max(logits, axis=-1, keepdims=True)
    p = jnp.exp(logits - m)
    p = p / jnp.sum(p, axis=-1, keepdims=True)
    i8 = lax.broadcasted_iota(jnp.int32, p.shape, 1)
    m1 = jnp.max(p, axis=-1, keepdims=True)
    i1 = jnp.min(jnp.where(p == m1, i8, E + 1), axis=-1, keepdims=True)
    p2 = jnp.where(i8 == i1, -1.0, p)
    m2 = jnp.max(p2, axis=-1, keepdims=True)
    i2 = jnp.min(jnp.where(p2 == m2, i8, E + 1), axis=-1, keepdims=True)
    s = m1 + m2
    topi_ref[...] = jnp.concatenate([i1, i2], axis=1)
    topv_ref[...] = jnp.concatenate([m1 / s, m2 / s], axis=1)


def _router(x2, gate_w):
    rb = 256
    return pl.pallas_call(
        _router_body,
        grid=(T // rb,),
        in_specs=[
            pl.BlockSpec((rb, D), lambda i: (i, 0)),
            pl.BlockSpec((E, D), lambda i: (0, 0)),
        ],
        out_specs=[
            pl.BlockSpec((rb, TK), lambda i: (i, 0)),
            pl.BlockSpec((rb, TK), lambda i: (i, 0)),
        ],
        out_shape=[
            jax.ShapeDtypeStruct((T, TK), jnp.int32),
            jax.ShapeDtypeStruct((T, TK), jnp.float32),
        ],
    )(x2, gate_w)


_XCH = 16

def _k2_body(topi_hbm, x_hbm, pos_hbm, blk_hbm, xs_hbm,
             buf, base_v, cur_v, pos_v, offblk_v, blk_v,
             xrows0, xrows1, idx_e0, idx_o0, idx_e1, idx_o1, sem):
    w = _wid()
    pltpu.sync_copy(topi_hbm, buf)
    zeros = jnp.zeros((16,), jnp.int32)
    ones = jnp.ones((16,), jnp.int32)
    base_v[...] = zeros
    cur_v[...] = zeros
    wstart = w * PW
    ii = lax.iota(jnp.int32, 16)
    for i in range(NPAIR // 16):
        v = buf[pl.ds(i * 16, 16)]
        plsc.addupdate_scatter(cur_v, [v], ones)
        inc = jnp.where(ii + i * 16 < wstart, 1, 0)
        plsc.addupdate_scatter(base_v, [v], inc)
    tot = cur_v[...]
    prior = base_v[...]
    pad = (tot + (BLK - 1)) & ~(BLK - 1)
    offpad = plsc.cumsum(pad) - pad
    base_v[...] = offpad + prior
    cur_v[...] = zeros
    for i in range(PW // 16):
        v = buf[pl.ds(wstart + i * 16, 16)]
        gb = plsc.load_gather(base_v, [v])
        gc = plsc.load_gather(cur_v, [v])
        plsc.addupdate_scatter(cur_v, [v], ones)
        rank = zeros
        for e in range(E):
            m = v == e
            r = plsc.cumsum(jnp.where(m, 1, 0))
            rank = jnp.where(m, r - 1, rank)
        pos_v[pl.ds(i * 16, 16)] = gb + gc + rank
    pltpu.sync_copy(pos_v, pos_hbm.at[pl.ds(wstart, PW)])

    @pl.when(w == 0)
    def _():
        offblk_v[...] = offpad // BLK
        for c in range(NBPAD // 16):
            bvec = lax.iota(jnp.int32, 16) + c * 16
            acc = jnp.zeros((16,), jnp.int32)
            for e in range(1, E):
                s_e = plsc.load_gather(offblk_v, [jnp.full((16,), e, jnp.int32)])
                acc = acc + jnp.where(bvec >= s_e, 1, 0)
            blk_v[pl.ds(c * 16, 16)] = acc
        pltpu.sync_copy(blk_v, blk_hbm)

    xrows = (xrows0, xrows1)
    idx_e = (idx_e0, idx_e1)
    idx_o = (idx_o0, idx_o1)
    nch = TW // _XCH
    cps = [None, None]

    def issue(c, s):
        tok0 = w * TW + c * _XCH
        pltpu.sync_copy(x_hbm.at[pl.ds(tok0, _XCH)], xrows[s])
        idx_e[s][...] = plsc.load_gather(pos_v, [c * 2 * _XCH + 2 * ii])
        idx_o[s][...] = plsc.load_gather(pos_v, [c * 2 * _XCH + 2 * ii + 1])
        return (pltpu.async_copy(xrows[s], xs_hbm.at[idx_e[s]], sem),
                pltpu.async_copy(xrows[s], xs_hbm.at[idx_o[s]], sem))

    cps[0] = issue(0, 0)
    for c in range(1, nch):
        s = c % 2
        cps[s] = issue(c, s)
        cps[1 - s][0].wait()
        cps[1 - s][1].wait()
    cps[(nch - 1) % 2][0].wait()
    cps[(nch - 1) % 2][1].wait()


def _dispatch_scatter(topi_flat, x2):
    return pl.kernel(
        _k2_body,
        out_type=[
            jax.ShapeDtypeStruct((NPAIR,), jnp.int32),
            jax.ShapeDtypeStruct((NBPAD,), jnp.int32),
            jax.ShapeDtypeStruct((P, D), jnp.float32),
        ],
        mesh=_MESH,
        compiler_params=pltpu.CompilerParams(needs_layout_passes=False),
        scratch_types=[
            pltpu.VMEM((NPAIR,), jnp.int32),
            pltpu.VMEM((16,), jnp.int32),
            pltpu.VMEM((16,), jnp.int32),
            pltpu.VMEM((PW,), jnp.int32),
            pltpu.VMEM((16,), jnp.int32),
            pltpu.VMEM((NBPAD,), jnp.int32),
            pltpu.VMEM((_XCH, D), jnp.float32),
            pltpu.VMEM((_XCH, D), jnp.float32),
            pltpu.VMEM((16,), jnp.int32),
            pltpu.VMEM((16,), jnp.int32),
            pltpu.VMEM((16,), jnp.int32),
            pltpu.VMEM((16,), jnp.int32),
            pltpu.SemaphoreType.DMA,
        ],
    )(topi_flat, x2)


def _ffn_body(be_ref, xs_ref, wg_ref, w1_ref, w2_ref, eo_ref):
    xb = xs_ref[...]
    g = lax.dot_general(xb, wg_ref[0], (((1,), (1,)), ((), ())),
                        preferred_element_type=jnp.float32)
    a = lax.dot_general(xb, w1_ref[0], (((1,), (1,)), ((), ())),
                        preferred_element_type=jnp.float32)
    h = a * (g * jax.nn.sigmoid(g))
    eo_ref[...] = lax.dot_general(h, w2_ref[0], (((1,), (1,)), ((), ())),
                                  preferred_element_type=jnp.float32)


def _ffn(blkexp, xs, wg, w1, w2):
    grid_spec = pltpu.PrefetchScalarGridSpec(
        num_scalar_prefetch=1,
        grid=(NB,),
        in_specs=[
            pl.BlockSpec((BLK, D), lambda b, be: (b, 0)),
            pl.BlockSpec((1, H, D), lambda b, be: (be[b], 0, 0)),
            pl.BlockSpec((1, H, D), lambda b, be: (be[b], 0, 0)),
            pl.BlockSpec((1, D, H), lambda b, be: (be[b], 0, 0)),
        ],
        out_specs=pl.BlockSpec((BLK, D), lambda b, be: (b, 0)),
    )
    return pl.pallas_call(
        _ffn_body,
        grid_spec=grid_spec,
        out_shape=jax.ShapeDtypeStruct((P, D), jnp.float32),
        compiler_params=pltpu.CompilerParams(
            dimension_semantics=("arbitrary",),
            vmem_limit_bytes=100 * 1024 * 1024),
    )(blkexp, xs, wg, w1, w2)


_CCH = 8

def _comb_body(eo_hbm, pos_hbm, tv_hbm, out_hbm, pos_v, tv_v, idx, rows, orow,
               sem):
    w = _wid()
    pltpu.sync_copy(pos_hbm.at[pl.ds(w * PW, PW)], pos_v)
    pltpu.sync_copy(tv_hbm.at[pl.ds(w * PW, PW)], tv_v)
    zero16 = jnp.zeros((16,), jnp.float32)
    for c in range(TW // _CCH):
        idx[...] = pos_v[pl.ds(c * 2 * _CCH, 2 * _CCH)]
        pltpu.async_copy(eo_hbm.at[idx], rows, sem).wait()
        wv = tv_v[pl.ds(c * 2 * _CCH, 2 * _CCH)]
        wb = [wv[j] + zero16 for j in range(2 * _CCH)]

        def body_r(r, _):
            off = r * 16
            for i in range(_CCH):
                a = rows[2 * i, pl.ds(off, 16)]
                b = rows[2 * i + 1, pl.ds(off, 16)]
                orow[i, pl.ds(off, 16)] = a * wb[2 * i] + b * wb[2 * i + 1]
            return 0

        lax.fori_loop(0, D // 16, body_r, 0)
        pltpu.sync_copy(orow, out_hbm.at[pl.ds(w * TW + c * _CCH, _CCH)])


def _combine(eo, pos, tv_flat):
    return pl.kernel(
        _comb_body,
        out_type=jax.ShapeDtypeStruct((T, D), jnp.float32),
        mesh=_MESH,
        compiler_params=pltpu.CompilerParams(needs_layout_passes=False),
        scratch_types=[
            pltpu.VMEM((PW,), jnp.int32),
            pltpu.VMEM((PW,), jnp.float32),
            pltpu.VMEM((2 * _CCH,), jnp.int32),
            pltpu.VMEM((2 * _CCH, D), jnp.float32),
            pltpu.VMEM((_CCH, D), jnp.float32),
            pltpu.SemaphoreType.DMA,
        ],
    )(eo, pos, tv_flat)


def kernel(x, gate_w, w1, wg, w2):
    x2 = x.reshape(T, D)
    topi, topv = _router(x2, gate_w)
    topi_flat = topi.reshape(NPAIR)
    tv_flat = topv.reshape(NPAIR)
    pos, blkexp, xs = _dispatch_scatter(topi_flat, x2)
    eo = _ffn(blkexp, xs, wg, w1, w2)
    out2 = _combine(eo, pos, tv_flat)
    return out2.reshape(B, S, D)


# speedup vs baseline: 1.0599x; 1.0007x over previous
import jax
import jax.numpy as jnp
from jax import lax
from jax.experimental import pallas as pl
from jax.experimental.pallas import tpu as pltpu
from jax.experimental.pallas import tpu_sc as plsc

B, S, D = 2, 2048, 2048
E, TK, H = 8, 2, 1024
T = B * S
NPAIR = T * TK
BLK = 256
P = NPAIR + E * BLK
NB = P // BLK
NBPAD = 48

NC, NS = 2, 16
NW = NC * NS
PW = NPAIR // NW
TW = T // NW

_MESH = plsc.VectorSubcoreMesh(
    core_axis_name="c", subcore_axis_name="s", num_cores=NC, num_subcores=NS)


def _wid():
    return lax.axis_index("s") * NC + lax.axis_index("c")


def _router_body(x_ref, g_ref, topi_ref, topv_ref):
    xb = x_ref[...]
    gw = g_ref[...]
    logits = lax.dot_general(xb, gw, (((1,), (1,)), ((), ())),
                             preferred_element_type=jnp.float32)
    m = jnp.max(logits, axis=-1, keepdims=True)
    p = jnp.exp(logits - m)
    p = p / jnp.sum(p, axis=-1, keepdims=True)
    i8 = lax.broadcasted_iota(jnp.int32, p.shape, 1)
    m1 = jnp.max(p, axis=-1, keepdims=True)
    i1 = jnp.min(jnp.where(p == m1, i8, E + 1), axis=-1, keepdims=True)
    p2 = jnp.where(i8 == i1, -1.0, p)
    m2 = jnp.max(p2, axis=-1, keepdims=True)
    i2 = jnp.min(jnp.where(p2 == m2, i8, E + 1), axis=-1, keepdims=True)
    s = m1 + m2
    topi_ref[...] = jnp.concatenate([i1, i2], axis=1)
    topv_ref[...] = jnp.concatenate([m1 / s, m2 / s], axis=1)


def _router(x2, gate_w):
    rb = 256
    return pl.pallas_call(
        _router_body,
        grid=(T // rb,),
        in_specs=[
            pl.BlockSpec((rb, D), lambda i: (i, 0)),
            pl.BlockSpec((E, D), lambda i: (0, 0)),
        ],
        out_specs=[
            pl.BlockSpec((rb, TK), lambda i: (i, 0)),
            pl.BlockSpec((rb, TK), lambda i: (i, 0)),
        ],
        out_shape=[
            jax.ShapeDtypeStruct((T, TK), jnp.int32),
            jax.ShapeDtypeStruct((T, TK), jnp.float32),
        ],
    )(x2, gate_w)


_XCH = 16

def _k2_body(topi_hbm, x_hbm, pos_hbm, blk_hbm, xs_hbm,
             buf, base_v, cur_v, pos_v, offblk_v, blk_v,
             xrows0, xrows1, idx_e0, idx_o0, idx_e1, idx_o1, sem):
    w = _wid()
    pltpu.sync_copy(topi_hbm, buf)
    zeros = jnp.zeros((16,), jnp.int32)
    ones = jnp.ones((16,), jnp.int32)
    base_v[...] = zeros
    cur_v[...] = zeros
    wstart = w * PW
    ii = lax.iota(jnp.int32, 16)
    for i in range(NPAIR // 16):
        v = buf[pl.ds(i * 16, 16)]
        plsc.addupdate_scatter(cur_v, [v], ones)
        inc = jnp.where(ii + i * 16 < wstart, 1, 0)
        plsc.addupdate_scatter(base_v, [v], inc)
    tot = cur_v[...]
    prior = base_v[...]
    pad = (tot + (BLK - 1)) & ~(BLK - 1)
    offpad = plsc.cumsum(pad) - pad
    base_v[...] = offpad + prior
    cur_v[...] = zeros
    for i in range(PW // 16):
        v = buf[pl.ds(wstart + i * 16, 16)]
        gb = plsc.load_gather(base_v, [v])
        gc = plsc.load_gather(cur_v, [v])
        plsc.addupdate_scatter(cur_v, [v], ones)
        rank = zeros
        for e in range(E):
            m = v == e
            r = plsc.cumsum(jnp.where(m, 1, 0))
            rank = jnp.where(m, r - 1, rank)
        pos_v[pl.ds(i * 16, 16)] = gb + gc + rank
    pltpu.sync_copy(pos_v, pos_hbm.at[pl.ds(wstart, PW)])

    @pl.when(w == 0)
    def _():
        offblk_v[...] = offpad // BLK
        for c in range(NBPAD // 16):
            bvec = lax.iota(jnp.int32, 16) + c * 16
            acc = jnp.zeros((16,), jnp.int32)
            for e in range(1, E):
                s_e = plsc.load_gather(offblk_v, [jnp.full((16,), e, jnp.int32)])
                acc = acc + jnp.where(bvec >= s_e, 1, 0)
            blk_v[pl.ds(c * 16, 16)] = acc
        pltpu.sync_copy(blk_v, blk_hbm)

    xrows = (xrows0, xrows1)
    idx_e = (idx_e0, idx_e1)
    idx_o = (idx_o0, idx_o1)
    nch = TW // _XCH
    cps = [None, None]

    def issue(c, s):
        tok0 = w * TW + c * _XCH
        pltpu.sync_copy(x_hbm.at[pl.ds(tok0, _XCH)], xrows[s])
        idx_e[s][...] = plsc.load_gather(pos_v, [c * 2 * _XCH + 2 * ii])
        idx_o[s][...] = plsc.load_gather(pos_v, [c * 2 * _XCH + 2 * ii + 1])
        return (pltpu.async_copy(xrows[s], xs_hbm.at[idx_e[s]], sem),
                pltpu.async_copy(xrows[s], xs_hbm.at[idx_o[s]], sem))

    cps[0] = issue(0, 0)
    for c in range(1, nch):
        s = c % 2
        cps[s] = issue(c, s)
        cps[1 - s][0].wait()
        cps[1 - s][1].wait()
    cps[(nch - 1) % 2][0].wait()
    cps[(nch - 1) % 2][1].wait()


def _dispatch_scatter(topi_flat, x2):
    return pl.kernel(
        _k2_body,
        out_type=[
            jax.ShapeDtypeStruct((NPAIR,), jnp.int32),
            jax.ShapeDtypeStruct((NBPAD,), jnp.int32),
            jax.ShapeDtypeStruct((P, D), jnp.float32),
        ],
        mesh=_MESH,
        compiler_params=pltpu.CompilerParams(needs_layout_passes=False),
        scratch_types=[
            pltpu.VMEM((NPAIR,), jnp.int32),
            pltpu.VMEM((16,), jnp.int32),
            pltpu.VMEM((16,), jnp.int32),
            pltpu.VMEM((PW,), jnp.int32),
            pltpu.VMEM((16,), jnp.int32),
            pltpu.VMEM((NBPAD,), jnp.int32),
            pltpu.VMEM((_XCH, D), jnp.float32),
            pltpu.VMEM((_XCH, D), jnp.float32),
            pltpu.VMEM((16,), jnp.int32),
            pltpu.VMEM((16,), jnp.int32),
            pltpu.VMEM((16,), jnp.int32),
            pltpu.VMEM((16,), jnp.int32),
            pltpu.SemaphoreType.DMA,
        ],
    )(topi_flat, x2)


def _ffn_body(be_ref, xs_ref, wg_ref, w1_ref, w2_ref, eo_ref):
    xb = xs_ref[...]
    g = lax.dot_general(xb, wg_ref[0], (((1,), (1,)), ((), ())),
                        preferred_element_type=jnp.float32)
    a = lax.dot_general(xb, w1_ref[0], (((1,), (1,)), ((), ())),
                        preferred_element_type=jnp.float32)
    h = a * (g * jax.nn.sigmoid(g))
    eo_ref[...] = lax.dot_general(h, w2_ref[0], (((1,), (1,)), ((), ())),
                                  preferred_element_type=jnp.float32)


def _ffn(blkexp, xs, wg, w1, w2):
    grid_spec = pltpu.PrefetchScalarGridSpec(
        num_scalar_prefetch=1,
        grid=(NB,),
        in_specs=[
            pl.BlockSpec((BLK, D), lambda b, be: (b, 0)),
            pl.BlockSpec((1, H, D), lambda b, be: (be[b], 0, 0)),
            pl.BlockSpec((1, H, D), lambda b, be: (be[b], 0, 0)),
            pl.BlockSpec((1, D, H), lambda b, be: (be[b], 0, 0)),
        ],
        out_specs=pl.BlockSpec((BLK, D), lambda b, be: (b, 0)),
    )
    return pl.pallas_call(
        _ffn_body,
        grid_spec=grid_spec,
        out_shape=jax.ShapeDtypeStruct((P, D), jnp.float32),
        compiler_params=pltpu.CompilerParams(
            dimension_semantics=("arbitrary",),
            vmem_limit_bytes=100 * 1024 * 1024),
    )(blkexp, xs, wg, w1, w2)


_CCH = 8

def _comb_body(eo_hbm, pos_hbm, tv_hbm, out_hbm, pos_v, tv_v, idx, rows, orow,
               sem):
    w = _wid()
    pltpu.sync_copy(pos_hbm.at[pl.ds(w * PW, PW)], pos_v)
    pltpu.sync_copy(tv_hbm.at[pl.ds(w * PW, PW)], tv_v)
    zero16 = jnp.zeros((16,), jnp.float32)
    for c in range(TW // _CCH):
        idx[...] = pos_v[pl.ds(c * 2 * _CCH, 2 * _CCH)]
        pltpu.async_copy(eo_hbm.at[idx], rows, sem).wait()
        wv = tv_v[pl.ds(c * 2 * _CCH, 2 * _CCH)]
        wb = [wv[j] + zero16 for j in range(2 * _CCH)]

        def body_r(r, _):
            off = r * 16
            for i in range(_CCH):
                a = rows[2 * i, pl.ds(off, 16)]
                b = rows[2 * i + 1, pl.ds(off, 16)]
                orow[i, pl.ds(off, 16)] = a * wb[2 * i] + b * wb[2 * i + 1]
            return 0

        lax.fori_loop(0, D // 16, body_r, 0)
        pltpu.sync_copy(orow, out_hbm.at[pl.ds(w * TW + c * _CCH, _CCH)])


def _combine(eo, pos, tv_flat):
    return pl.kernel(
        _comb_body,
        out_type=jax.ShapeDtypeStruct((T, D), jnp.float32),
        mesh=_MESH,
        compiler_params=pltpu.CompilerParams(needs_layout_passes=False),
        scratch_types=[
            pltpu.VMEM((PW,), jnp.int32),
            pltpu.VMEM((PW,), jnp.float32),
            pltpu.VMEM((2 * _CCH,), jnp.int32),
            pltpu.VMEM((2 * _CCH, D), jnp.float32),
            pltpu.VMEM((_CCH, D), jnp.float32),
            pltpu.SemaphoreType.DMA,
        ],
    )(eo, pos, tv_flat)


def kernel(x, gate_w, w1, wg, w2):
    x2 = x.reshape(T, D)
    topi, topv = _router(x2, gate_w)
    topi_flat = topi.reshape(NPAIR)
    tv_flat = topv.reshape(NPAIR)
    pos, blkexp, xs = _dispatch_scatter(topi_flat, x2)
    eo = _ffn(blkexp, xs, wg, w1, w2)
    out2 = _combine(eo, pos, tv_flat)
    return out2.reshape(B, S, D)
